# baseline (speedup 1.0000x reference)
import jax
import jax.numpy as jnp
from jax.experimental import pallas as pl
from jax.experimental.pallas import tpu as pltpu

_N, _S, _C, _K, _D = 8, 576, 1, 1024, 256
_NS = _N * _S
_R = 768
_G = _NS // _R


def _vq_kernel(x_ref, e_ref, out0_ref, out1_ref, hist_ref, ent_ref,
               e1_ref, e2_ref, e3_ref):
    i = pl.program_id(0)
    x = x_ref[...]

    @pl.when(i == 0)
    def _():
        e = e_ref[...]
        e1_ref[...] = e.astype(jnp.bfloat16)
        e2r = e - e1_ref[...].astype(jnp.float32)
        e2_ref[...] = e2r.astype(jnp.bfloat16)
        e3_ref[...] = (e2r - e2_ref[...].astype(jnp.float32)).astype(
            jnp.bfloat16)

    e1 = e1_ref[...]
    e2 = e2_ref[...]
    e3 = e3_ref[...]

    xe = jax.lax.dot_general(
        x.astype(jnp.bfloat16), e1,
        (((1,), (1,)), ((), ())), preferred_element_type=jnp.float32)
    s = -2.0 * xe

    iota = jax.lax.broadcasted_iota(jnp.int32, (_R, _K), 1)
    big = jnp.float32(3.0e38)

    cand = []
    for _ in range(3):
        m = jnp.min(s, axis=1)[:, None]
        ij = jnp.min(jnp.where(s == m, iota, _K), axis=1)[:, None]
        cand.append(ij)
        s = jnp.where(iota == ij, big, s)

    best_d = best_i = best_e = None
    for ij in cand:
        oh = (iota == ij).astype(jnp.bfloat16)
        dn = (((1,), (0,)), ((), ()))
        ev = (jax.lax.dot_general(oh, e1, dn, preferred_element_type=jnp.float32)
              + jax.lax.dot_general(oh, e2, dn, preferred_element_type=jnp.float32)
              ) + jax.lax.dot_general(oh, e3, dn, preferred_element_type=jnp.float32)
        diff = x - ev
        sq = diff * diff
        dp = sq[:, :128] + sq[:, 128:]
        acc = dp
        for j in range(1, 16):
            acc = acc + jnp.roll(dp, -8 * j, axis=1)
        b = acc + jnp.roll(acc, -4, axis=1)
        c = b + jnp.roll(b, -2, axis=1)
        t = c + jnp.roll(c, -1, axis=1)
        d2 = t[:, 0:1]
        d = d2 * jax.lax.rsqrt(d2)
        if best_d is None:
            best_d, best_i, best_e = d, ij, ev
        else:
            take = (d < best_d) | ((d == best_d) & (ij < best_i))
            best_d = jnp.where(take, d, best_d)
            best_i = jnp.where(take, ij, best_i)
            best_e = jnp.where(take, ev, best_e)

    out0_ref[...] = (best_e - x) + x
    out1_ref[...] = best_d * best_d

    ohw = (iota == best_i).astype(jnp.float32)
    h = jnp.sum(ohw, axis=0, keepdims=True)

    @pl.when(i == 0)
    def _():
        hist_ref[...] = h

    @pl.when(i != 0)
    def _():
        hist_ref[...] = hist_ref[...] + h

    @pl.when(i == _G - 1)
    def _():
        hist = hist_ref[...]
        prob = hist / jnp.float32(_NS)
        ent = -jnp.sum(jnp.where(hist > 0,
                                 prob * jnp.log(jnp.where(hist > 0, prob, 1.0)),
                                 0.0))
        ent_ref[...] = ent.reshape(1, 1)


def _vq(x2, e2):
    out_shapes = (
        jax.ShapeDtypeStruct((_NS, _D), jnp.float32),
        jax.ShapeDtypeStruct((_NS, 1), jnp.float32),
        jax.ShapeDtypeStruct((1, _K), jnp.float32),
        jax.ShapeDtypeStruct((1, 1), jnp.float32),
    )
    return pl.pallas_call(
        _vq_kernel,
        grid=(_G,),
        in_specs=[
            pl.BlockSpec((_R, _D), lambda i: (i, 0)),
            pl.BlockSpec((_K, _D), lambda i: (0, 0)),
        ],
        out_specs=(
            pl.BlockSpec((_R, _D), lambda i: (i, 0)),
            pl.BlockSpec((_R, 1), lambda i: (i, 0)),
            pl.BlockSpec((1, _K), lambda i: (0, 0)),
            pl.BlockSpec((1, 1), lambda i: (0, 0)),
        ),
        out_shape=out_shapes,
        scratch_shapes=[pltpu.VMEM((_K, _D), jnp.bfloat16)] * 3,
        compiler_params=pltpu.CompilerParams(
            dimension_semantics=("arbitrary",)),
    )(x2, e2)


def kernel(x0, embedding0):
    x2 = x0.reshape(_NS, _D)
    e2 = embedding0.reshape(_K, _D)
    o0, o1, _hist, ent = _vq(x2, e2)
    out0 = o0.reshape(_N, _S, _C, _D)
    out1 = o1.reshape(_N, _S, _C)
    entropy = ent[0, 0]
    return (out0, out1, out1, entropy)


# speedup vs baseline: 1.2863x; 1.2863x over previous
import jax
import jax.numpy as jnp
from jax.experimental import pallas as pl
from jax.experimental.pallas import tpu as pltpu

_N, _S, _C, _K, _D = 8, 576, 1, 1024, 256
_NS = _N * _S
_R = 768
_G = _NS // _R


def _vq_kernel(x_ref, e_ref, out0_ref, out1_ref, hist_ref, ent_ref,
               e1_ref, e2_ref, e3_ref):
    i = pl.program_id(0)
    x = x_ref[...]

    @pl.when(i == 0)
    def _():
        e = e_ref[...]
        e1_ref[...] = e.astype(jnp.bfloat16)
        e2r = e - e1_ref[...].astype(jnp.float32)
        e2_ref[...] = e2r.astype(jnp.bfloat16)
        e3_ref[...] = (e2r - e2_ref[...].astype(jnp.float32)).astype(
            jnp.bfloat16)

    e1 = e1_ref[...]
    e2 = e2_ref[...]
    e3 = e3_ref[...]

    xe = jax.lax.dot_general(
        x.astype(jnp.bfloat16), e1,
        (((1,), (1,)), ((), ())), preferred_element_type=jnp.float32)
    s = -2.0 * xe

    iota = jax.lax.broadcasted_iota(jnp.int32, (_R, _K), 1)
    big = jnp.float32(3.0e38)

    cand = []
    for _ in range(3):
        m = jnp.min(s, axis=1)[:, None]
        ij = jnp.min(jnp.where(s == m, iota, _K), axis=1)[:, None]
        cand.append(ij)
        s = jnp.where(iota == ij, big, s)

    best_d = best_i = best_e = None
    for ij in cand:
        oh = (iota == ij).astype(jnp.bfloat16)
        dn = (((1,), (0,)), ((), ()))
        ev = (jax.lax.dot_general(oh, e1, dn, preferred_element_type=jnp.float32)
              + jax.lax.dot_general(oh, e2, dn, preferred_element_type=jnp.float32)
              ) + jax.lax.dot_general(oh, e3, dn, preferred_element_type=jnp.float32)
        diff = x - ev
        sq = diff * diff
        dp = sq[:, :128] + sq[:, 128:]
        dpt = jnp.transpose(dp)
        acc = dpt[0:8, :]
        for j in range(1, 16):
            acc = acc + dpt[8 * j:8 * j + 8, :]
        b = acc[0:4, :] + acc[4:8, :]
        c = b[0:2, :] + b[2:4, :]
        t = c[0:1, :] + c[1:2, :]
        d2 = jnp.transpose(t)
        d = d2 * jax.lax.rsqrt(d2)
        if best_d is None:
            best_d, best_i, best_e = d, ij, ev
        else:
            take = (d < best_d) | ((d == best_d) & (ij < best_i))
            best_d = jnp.where(take, d, best_d)
            best_i = jnp.where(take, ij, best_i)
            best_e = jnp.where(take, ev, best_e)

    out0_ref[...] = (best_e - x) + x
    out1_ref[...] = best_d * best_d

    ohw = (iota == best_i).astype(jnp.float32)
    h = jnp.sum(ohw, axis=0, keepdims=True)

    @pl.when(i == 0)
    def _():
        hist_ref[...] = h

    @pl.when(i != 0)
    def _():
        hist_ref[...] = hist_ref[...] + h

    @pl.when(i == _G - 1)
    def _():
        hist = hist_ref[...]
        prob = hist / jnp.float32(_NS)
        ent = -jnp.sum(jnp.where(hist > 0,
                                 prob * jnp.log(jnp.where(hist > 0, prob, 1.0)),
                                 0.0))
        ent_ref[...] = ent.reshape(1, 1)


def _vq(x2, e2):
    out_shapes = (
        jax.ShapeDtypeStruct((_NS, _D), jnp.float32),
        jax.ShapeDtypeStruct((_NS, 1), jnp.float32),
        jax.ShapeDtypeStruct((1, _K), jnp.float32),
        jax.ShapeDtypeStruct((1, 1), jnp.float32),
    )
    return pl.pallas_call(
        _vq_kernel,
        grid=(_G,),
        in_specs=[
            pl.BlockSpec((_R, _D), lambda i: (i, 0)),
            pl.BlockSpec((_K, _D), lambda i: (0, 0)),
        ],
        out_specs=(
            pl.BlockSpec((_R, _D), lambda i: (i, 0)),
            pl.BlockSpec((_R, 1), lambda i: (i, 0)),
            pl.BlockSpec((1, _K), lambda i: (0, 0)),
            pl.BlockSpec((1, 1), lambda i: (0, 0)),
        ),
        out_shape=out_shapes,
        scratch_shapes=[pltpu.VMEM((_K, _D), jnp.bfloat16)] * 3,
        compiler_params=pltpu.CompilerParams(
            dimension_semantics=("arbitrary",)),
    )(x2, e2)


def kernel(x0, embedding0):
    x2 = x0.reshape(_NS, _D)
    e2 = embedding0.reshape(_K, _D)
    o0, o1, _hist, ent = _vq(x2, e2)
    out0 = o0.reshape(_N, _S, _C, _D)
    out1 = o1.reshape(_N, _S, _C)
    entropy = ent[0, 0]
    return (out0, out1, out1, entropy)
